# baseline (speedup 1.0000x reference)
import functools

import jax
import jax.numpy as jnp
from jax import lax
from jax.experimental import pallas as pl
from jax.experimental.pallas import tpu as pltpu
from jax.experimental.pallas import tpu_sc as plsc

VOCAB = 1000000
EMBED = 64
LANES = 128
BATCH = 4096
WINDOW = 200

TOTAL = 2 * BATCH * WINDOW
NUM_CORES = 2
NUM_SUBCORES = 16
NW = NUM_CORES * NUM_SUBCORES
PER_W = TOTAL // NW
CHUNK = 256
NBUF = 4
GROUP = NBUF * CHUNK
NGROUP = PER_W // GROUP

_mesh = plsc.VectorSubcoreMesh(
    core_axis_name="c", subcore_axis_name="s",
    num_cores=NUM_CORES, num_subcores=NUM_SUBCORES,
)


HALF = EMBED // 2


@functools.partial(
    pl.kernel,
    out_type=jax.ShapeDtypeStruct((TOTAL, LANES), jnp.float32),
    mesh=_mesh,
    scratch_types=[
        pltpu.VMEM((PER_W,), jnp.int32),
        pltpu.VMEM((NBUF, CHUNK, HALF), jnp.float32),
        pltpu.VMEM((NBUF, CHUNK, HALF), jnp.float32),
        pltpu.SemaphoreType.DMA,
        pltpu.SemaphoreType.DMA,
    ],
    compiler_params=pltpu.CompilerParams(use_tc_tiling_on_sc=False),
)
def _gather_all(ta_hbm, tb_hbm, idx_hbm, out_hbm, idx_v, rows_a, rows_b,
                sem_g, sem_s):
    wid = lax.axis_index("s") * NUM_CORES + lax.axis_index("c")
    base = wid * PER_W
    pltpu.sync_copy(idx_hbm.at[pl.ds(base, PER_W)], idx_v)

    def group(g, carry):
        goff = g * GROUP
        gat = []
        for b in range(NBUF):
            isl = idx_v.at[pl.ds(goff + b * CHUNK, CHUNK)]
            gat.append((
                pltpu.async_copy(ta_hbm.at[isl], rows_a.at[b], sem_g),
                pltpu.async_copy(tb_hbm.at[isl], rows_b.at[b], sem_g),
            ))
        sto = []
        for b in range(NBUF):
            gat[b][0].wait()
            gat[b][1].wait()
            osl = pl.ds(base + goff + b * CHUNK, CHUNK)
            sto.append(pltpu.async_copy(
                rows_a.at[b], out_hbm.at[osl, pl.ds(0, HALF)], sem_s))
            sto.append(pltpu.async_copy(
                rows_b.at[b], out_hbm.at[osl, pl.ds(HALF, HALF)], sem_s))
        for d in sto:
            d.wait()
        return carry

    lax.fori_loop(0, NGROUP, group, 0)


def kernel(inputs, labels, E):
    idx = jnp.concatenate(
        [inputs.reshape(-1), labels.reshape(-1)]).astype(jnp.int32)
    out = _gather_all(E[:, :HALF], E[:, HALF:], idx)
    return out[:, :EMBED].reshape(2, BATCH, WINDOW, EMBED)


# speedup vs baseline: 1.6468x; 1.6468x over previous
import functools

import jax
import jax.numpy as jnp
from jax import lax
from jax.experimental import pallas as pl
from jax.experimental.pallas import tpu as pltpu
from jax.experimental.pallas import tpu_sc as plsc

VOCAB = 1000000
EMBED = 64
LANES = 128
BATCH = 4096
WINDOW = 200

TOTAL = 2 * BATCH * WINDOW
NUM_CORES = 2
NUM_SUBCORES = 16
NW = NUM_CORES * NUM_SUBCORES
PER_W = TOTAL // NW
CHUNK = 256
NBUF = 4
GROUP = NBUF * CHUNK
NGROUP = PER_W // GROUP

_mesh = plsc.VectorSubcoreMesh(
    core_axis_name="c", subcore_axis_name="s",
    num_cores=NUM_CORES, num_subcores=NUM_SUBCORES,
)


@functools.partial(
    pl.kernel,
    out_type=jax.ShapeDtypeStruct((TOTAL, LANES), jnp.float32),
    mesh=_mesh,
    scratch_types=[
        pltpu.VMEM((PER_W,), jnp.int32),
        pltpu.VMEM((NBUF, CHUNK, EMBED), jnp.float32),
        pltpu.SemaphoreType.DMA,
        pltpu.SemaphoreType.DMA,
    ],
    compiler_params=pltpu.CompilerParams(use_tc_tiling_on_sc=False),
)
def _gather_all(table_hbm, idx_hbm, out_hbm, idx_v, rows_v, sem_g, sem_s):
    wid = lax.axis_index("s") * NUM_CORES + lax.axis_index("c")
    base = wid * PER_W
    pltpu.sync_copy(idx_hbm.at[pl.ds(base, PER_W)], idx_v)

    def group(g, carry):
        goff = g * GROUP
        gat = [
            pltpu.async_copy(
                table_hbm.at[idx_v.at[pl.ds(goff + b * CHUNK, CHUNK)]],
                rows_v.at[b], sem_g)
            for b in range(NBUF)
        ]
        sto = []
        for b in range(NBUF):
            gat[b].wait()
            sto.append(pltpu.async_copy(
                rows_v.at[b],
                out_hbm.at[pl.ds(base + goff + b * CHUNK, CHUNK),
                           pl.ds(0, EMBED)], sem_s))
        for d in sto:
            d.wait()
        return carry

    lax.fori_loop(0, NGROUP, group, 0)


def kernel(inputs, labels, E):
    idx = jnp.concatenate(
        [inputs.reshape(-1), labels.reshape(-1)]).astype(jnp.int32)
    out = _gather_all(E, idx)
    return out[:, :EMBED].reshape(2, BATCH, WINDOW, EMBED)
